# baseline (speedup 1.0000x reference)
import jax
import jax.numpy as jnp
from jax import lax
from jax.experimental import pallas as pl
from jax.experimental.pallas import tpu as pltpu
from jax.experimental.pallas import tpu_sc as plsc

N = 10000
E = 320000
D = 128
D_EDGE = 16

NC = 2
NS = 16
NW = NC * NS
EPW = E // NW
B = 40
NB = EPW // B
N_PAD = 10240
RPS = N_PAD // NS
CR = N_PAD // D

_LEAKY = 0.01


def _leaky(v):
    return jnp.where(v >= 0.0, v, v * _LEAKY)


def _proj_body(x_ref, ws_ref, wd_ref, xs_ref, xd_ref):
    xb = x_ref[...]
    xs_ref[...] = jnp.dot(xb, ws_ref[...], preferred_element_type=jnp.float32)
    xd_ref[...] = jnp.dot(xb, wd_ref[...], preferred_element_type=jnp.float32)


def _proj(x, w_s, w_d):
    nb = 25
    bn = N // nb
    return pl.pallas_call(
        _proj_body,
        grid=(nb,),
        in_specs=[
            pl.BlockSpec((bn, D), lambda i: (i, 0)),
            pl.BlockSpec((D, D), lambda i: (0, 0)),
            pl.BlockSpec((D, D), lambda i: (0, 0)),
        ],
        out_specs=[pl.BlockSpec((bn, D), lambda i: (i, 0))] * 2,
        out_shape=[jax.ShapeDtypeStruct((N, D), jnp.float32)] * 2,
    )(x, w_s, w_d)


def _ea_body(e_ref, w_ref, o_ref):
    o_ref[...] = jnp.dot(e_ref[...], w_ref[...], preferred_element_type=jnp.float32)


def _ea_proj(edge_attr, w_e):
    nb = 40
    be = E // nb
    return pl.pallas_call(
        _ea_body,
        grid=(nb,),
        in_specs=[
            pl.BlockSpec((be, D_EDGE), lambda i: (i, 0)),
            pl.BlockSpec((D_EDGE, D), lambda i: (0, 0)),
        ],
        out_specs=pl.BlockSpec((be, D), lambda i: (i, 0)),
        out_shape=jax.ShapeDtypeStruct((E, D), jnp.float32),
    )(edge_attr, w_e)


def _edge_kernel(src_hbm, dst_hbm, xs_hbm, xd_hbm, ea_hbm,
                 out_msg, out_cnt,
                 src_t, dst_t, dst_v, xs_v, xd_v, msg_v, cnt_v,
                 acc_sh, sem_g1, sem_g2, sem_ea, sem_sc):
    c = lax.axis_index("c")
    s = lax.axis_index("s")
    wid = c * NS + s

    def _zero_cnt(i, carry):
        for j in range(D // 16):
            cnt_v[i, pl.ds(j * 16, 16)] = jnp.zeros((16,), jnp.float32)
        return carry
    lax.fori_loop(0, CR, _zero_cnt, 0)

    def _zero_xs(i, carry):
        for j in range(D // 16):
            xs_v[i, pl.ds(j * 16, 16)] = jnp.zeros((16,), jnp.float32)
        return carry
    lax.fori_loop(0, B, _zero_xs, 0)

    def _zinit(k, carry):
        pltpu.sync_copy(xs_v, acc_sh.at[pl.ds(s * RPS + k * B, B)])
        return carry
    lax.fori_loop(0, RPS // B, _zinit, 0)

    base0 = wid * EPW
    pltpu.sync_copy(src_hbm.at[pl.ds(base0, EPW)], src_t)
    pltpu.sync_copy(dst_hbm.at[pl.ds(base0, EPW)], dst_t)

    plsc.subcore_barrier()

    ones16 = jnp.ones((16,), jnp.float32)
    tail_mask = lax.iota(jnp.int32, 16) >= 8

    pltpu.async_copy(xs_hbm.at[src_t.at[pl.ds(0, B)]], xs_v, sem_g1)
    pltpu.async_copy(xd_hbm.at[dst_t.at[pl.ds(0, B)]], xd_v, sem_g2)
    pltpu.async_copy(ea_hbm.at[pl.ds(base0, B)], msg_v, sem_ea)

    def _block(b, carry):
        eb = b * B
        pltpu.make_async_copy(xs_hbm.at[src_t.at[pl.ds(0, B)]], xs_v,
                              sem_g1).wait()
        pltpu.make_async_copy(xd_hbm.at[dst_t.at[pl.ds(0, B)]], xd_v,
                              sem_g2).wait()
        pltpu.make_async_copy(ea_hbm.at[pl.ds(0, B)], msg_v, sem_ea).wait()

        for off in (0, 16, 24):
            dst_v[pl.ds(off, 16)] = dst_t[pl.ds(eb + off, 16)]

        @plsc.parallel_loop(0, B, unroll=2)
        def _row(i):
            for j in range(D // 16):
                sl = pl.ds(j * 16, 16)
                v = xs_v[i, sl] + xd_v[i, sl] + msg_v[i, sl]
                msg_v[i, sl] = _leaky(v)

        @pl.when(b < NB - 1)
        def _pref():
            nb = eb + B
            pltpu.async_copy(xs_hbm.at[src_t.at[pl.ds(nb, B)]], xs_v, sem_g1)
            pltpu.async_copy(xd_hbm.at[dst_t.at[pl.ds(nb, B)]], xd_v, sem_g2)

        sc_cp = pltpu.async_copy(msg_v, acc_sh.at[dst_v], sem_sc, add=True)

        for off, msk in ((0, None), (16, None), (24, tail_mask)):
            idx = dst_v[pl.ds(off, 16)]
            hi = lax.shift_right_logical(idx, 7)
            lo = lax.bitwise_and(idx, 127)
            plsc.addupdate_scatter(cnt_v, (hi, lo), ones16, mask=msk)

        sc_cp.wait()

        @pl.when(b < NB - 1)
        def _pref_ea():
            pltpu.async_copy(ea_hbm.at[pl.ds(base0 + eb + B, B)],
                             msg_v, sem_ea)
        return carry
    lax.fori_loop(0, NB, _block, 0)

    plsc.subcore_barrier()

    def _wout(k, carry):
        row = s * RPS + k * B
        pltpu.sync_copy(acc_sh.at[pl.ds(row, B)], xs_v)
        pltpu.sync_copy(xs_v, out_msg.at[c, pl.ds(row, B)])
        return carry
    lax.fori_loop(0, RPS // B, _wout, 0)
    pltpu.sync_copy(cnt_v.at[pl.ds(0, CR)], out_cnt.at[wid])


_edge_call = pl.kernel(
    _edge_kernel,
    out_type=(jax.ShapeDtypeStruct((NC, N_PAD, D), jnp.float32),
              jax.ShapeDtypeStruct((NW, CR, D), jnp.float32)),
    mesh=plsc.VectorSubcoreMesh(core_axis_name="c", subcore_axis_name="s"),
    compiler_params=pltpu.CompilerParams(needs_layout_passes=False),
    scratch_types=[
        pltpu.VMEM((EPW,), jnp.int32),
        pltpu.VMEM((EPW,), jnp.int32),
        pltpu.VMEM((B,), jnp.int32),
        pltpu.VMEM((B, D), jnp.float32),
        pltpu.VMEM((B, D), jnp.float32),
        pltpu.VMEM((B, D), jnp.float32),
        pltpu.VMEM((CR, D), jnp.float32),
        pltpu.VMEM_SHARED((N_PAD, D), jnp.float32),
        pltpu.SemaphoreType.DMA,
        pltpu.SemaphoreType.DMA,
        pltpu.SemaphoreType.DMA,
        pltpu.SemaphoreType.DMA,
    ],
)


def _fin_body(x_ref, mp_ref, cp_ref, wux_ref, wun_ref, o_ref):
    ssum = mp_ref[0] + mp_ref[1]
    cnt = jnp.sum(cp_ref[...], axis=0)[:, None]
    h = ssum / jnp.maximum(cnt, 1.0)
    y = (jnp.dot(x_ref[...], wux_ref[...], preferred_element_type=jnp.float32)
         + jnp.dot(h, wun_ref[...], preferred_element_type=jnp.float32))
    o_ref[...] = _leaky(y)


def _final(x_pad, msg_p, cnt_r, wu_x, wu_n):
    nb = 20
    bn = N_PAD // nb
    return pl.pallas_call(
        _fin_body,
        grid=(nb,),
        in_specs=[
            pl.BlockSpec((bn, D), lambda i: (i, 0)),
            pl.BlockSpec((NC, bn, D), lambda i: (0, i, 0)),
            pl.BlockSpec((NW, bn), lambda i: (0, i)),
            pl.BlockSpec((D, D), lambda i: (0, 0)),
            pl.BlockSpec((D, D), lambda i: (0, 0)),
        ],
        out_specs=pl.BlockSpec((bn, D), lambda i: (i, 0)),
        out_shape=jax.ShapeDtypeStruct((N_PAD, D), jnp.float32),
    )(x_pad, msg_p, cnt_r, wu_x, wu_n)


def kernel(x, edge_index, edge_attr, W_M, W_U):
    src = edge_index[0]
    dst = edge_index[1]
    w_s = W_M[:D]
    w_d = W_M[D:2 * D]
    w_e = W_M[2 * D:]
    wu_x = W_U[:D]
    wu_n = W_U[D:]

    xs, xd = _proj(x, w_s, w_d)
    ea = _ea_proj(edge_attr, w_e)

    msg_p, cnt_p = _edge_call(src, dst, xs, xd, ea)
    cnt_r = cnt_p.reshape(NW, N_PAD)

    x_pad = jnp.pad(x, ((0, N_PAD - N), (0, 0)))
    out = _final(x_pad, msg_p, cnt_r, wu_x, wu_n)
    return out[:N]


# speedup vs baseline: 1.1110x; 1.1110x over previous
import jax
import jax.numpy as jnp
from jax import lax
from jax.experimental import pallas as pl
from jax.experimental.pallas import tpu as pltpu
from jax.experimental.pallas import tpu_sc as plsc

N = 10000
E = 320000
D = 128
D_EDGE = 16

NC = 2
NS = 16
NW = NC * NS
EPW = E // NW
B = 80
NB = EPW // B
N_PAD = 10240
RPS = N_PAD // NS
CR = N_PAD // D
NPAIR = (NB - 1) // 2

_LEAKY = 0.01


def _leaky(v):
    return jnp.where(v >= 0.0, v, v * _LEAKY)


def _proj_body(x_ref, ws_ref, wd_ref, xs_ref, xd_ref):
    xb = x_ref[...]
    xs_ref[...] = jnp.dot(xb, ws_ref[...], preferred_element_type=jnp.float32)
    xd_ref[...] = jnp.dot(xb, wd_ref[...], preferred_element_type=jnp.float32)


def _proj(x, w_s, w_d):
    nb = 25
    bn = N // nb
    return pl.pallas_call(
        _proj_body,
        grid=(nb,),
        in_specs=[
            pl.BlockSpec((bn, D), lambda i: (i, 0)),
            pl.BlockSpec((D, D), lambda i: (0, 0)),
            pl.BlockSpec((D, D), lambda i: (0, 0)),
        ],
        out_specs=[pl.BlockSpec((bn, D), lambda i: (i, 0))] * 2,
        out_shape=[jax.ShapeDtypeStruct((N, D), jnp.float32)] * 2,
    )(x, w_s, w_d)


def _ea_body(e_ref, w_ref, o_ref):
    o_ref[...] = jnp.dot(e_ref[...], w_ref[...], preferred_element_type=jnp.float32)


def _ea_proj(edge_attr, w_e):
    nb = 40
    be = E // nb
    return pl.pallas_call(
        _ea_body,
        grid=(nb,),
        in_specs=[
            pl.BlockSpec((be, D_EDGE), lambda i: (i, 0)),
            pl.BlockSpec((D_EDGE, D), lambda i: (0, 0)),
        ],
        out_specs=pl.BlockSpec((be, D), lambda i: (i, 0)),
        out_shape=jax.ShapeDtypeStruct((E, D), jnp.float32),
    )(edge_attr, w_e)


def _edge_kernel(src_hbm, dst_hbm, xs_hbm, xd_hbm, ea_hbm,
                 out_msg, out_cnt,
                 si_a, di_a, si_b, di_b, xs_v, xd_v, msg_v, cnt_v,
                 acc_sh, sem_i, sem_g1, sem_g2, sem_ea, sem_sc):
    c = lax.axis_index("c")
    s = lax.axis_index("s")
    wid = c * NS + s

    def _zero_cnt(i, carry):
        for j in range(D // 16):
            cnt_v[i, pl.ds(j * 16, 16)] = jnp.zeros((16,), jnp.float32)
        return carry
    lax.fori_loop(0, CR, _zero_cnt, 0)

    def _zero_xs(i, carry):
        for j in range(D // 16):
            xs_v[i, pl.ds(j * 16, 16)] = jnp.zeros((16,), jnp.float32)
        return carry
    lax.fori_loop(0, B, _zero_xs, 0)

    def _zinit(k, carry):
        pltpu.sync_copy(xs_v, acc_sh.at[pl.ds(s * RPS + k * B, B)])
        return carry
    lax.fori_loop(0, RPS // B, _zinit, 0)

    base0 = wid * EPW
    plsc.subcore_barrier()

    ones16 = jnp.ones((16,), jnp.float32)

    def _wait_gathers():
        pltpu.make_async_copy(xs_hbm.at[si_a], xs_v, sem_g1).wait()
        pltpu.make_async_copy(xd_hbm.at[di_a], xd_v, sem_g2).wait()
        pltpu.make_async_copy(ea_hbm.at[pl.ds(0, B)], msg_v, sem_ea).wait()

    def _wait_idx(si, di):
        pltpu.make_async_copy(src_hbm.at[pl.ds(0, B)], si, sem_i).wait()
        pltpu.make_async_copy(dst_hbm.at[pl.ds(0, B)], di, sem_i).wait()

    def _compute():
        def _row(i, carry2):
            for j in range(D // 16):
                sl = pl.ds(j * 16, 16)
                v = xs_v[i, sl] + xd_v[i, sl] + msg_v[i, sl]
                msg_v[i, sl] = _leaky(v)
            return carry2
        lax.fori_loop(0, B, _row, 0)

    def _counts(di):
        for kk in range(B // 16):
            idx = di[pl.ds(kk * 16, 16)]
            hi = lax.shift_right_logical(idx, 7)
            lo = lax.bitwise_and(idx, 127)
            plsc.addupdate_scatter(cnt_v, (hi, lo), ones16)

    def _half(k, si_p, di_p, si_q, di_q, gate_next, gate_idx2,
              gate_widx=True):
        _wait_gathers()

        @pl.when(gate_widx)
        def _widx():
            _wait_idx(si_q, di_q)
        _compute()

        @pl.when(gate_next)
        def _pref():
            pltpu.async_copy(xs_hbm.at[si_q], xs_v, sem_g1)
            pltpu.async_copy(xd_hbm.at[di_q], xd_v, sem_g2)

        sc_cp = pltpu.async_copy(msg_v, acc_sh.at[di_p], sem_sc, add=True)
        _counts(di_p)
        sc_cp.wait()
        @pl.when(gate_idx2)
        def _idx2():
            eb2 = base0 + (k + 2) * B
            pltpu.async_copy(src_hbm.at[pl.ds(eb2, B)], si_p, sem_i)
            pltpu.async_copy(dst_hbm.at[pl.ds(eb2, B)], di_p, sem_i)

        @pl.when(gate_next)
        def _pref_ea():
            pltpu.async_copy(ea_hbm.at[pl.ds(base0 + (k + 1) * B, B)],
                             msg_v, sem_ea)

    pltpu.sync_copy(src_hbm.at[pl.ds(base0, B)], si_a)
    pltpu.sync_copy(dst_hbm.at[pl.ds(base0, B)], di_a)
    pltpu.async_copy(xs_hbm.at[si_a], xs_v, sem_g1)
    pltpu.async_copy(xd_hbm.at[di_a], xd_v, sem_g2)
    pltpu.async_copy(ea_hbm.at[pl.ds(base0, B)], msg_v, sem_ea)
    pltpu.async_copy(src_hbm.at[pl.ds(base0 + B, B)], si_b, sem_i)
    pltpu.async_copy(dst_hbm.at[pl.ds(base0 + B, B)], di_b, sem_i)

    _half(jnp.int32(0), si_a, di_a, si_b, di_b, True, True)

    def _pair(t, carry):
        k1 = 2 * t + 1
        _half(k1, si_b, di_b, si_a, di_a, True, t < NPAIR - 1)
        _half(k1 + 1, si_a, di_a, si_b, di_b, t < NPAIR - 1, t < NPAIR - 1,
              gate_widx=t < NPAIR - 1)
        return carry
    lax.fori_loop(0, NPAIR, _pair, 0)

    plsc.subcore_barrier()

    def _wout(k, carry):
        row = s * RPS + k * B
        pltpu.sync_copy(acc_sh.at[pl.ds(row, B)], xs_v)
        pltpu.sync_copy(xs_v, out_msg.at[c, pl.ds(row, B)])
        return carry
    lax.fori_loop(0, RPS // B, _wout, 0)
    pltpu.sync_copy(cnt_v.at[pl.ds(0, CR)], out_cnt.at[wid])


_edge_call = pl.kernel(
    _edge_kernel,
    out_type=(jax.ShapeDtypeStruct((NC, N_PAD, D), jnp.float32),
              jax.ShapeDtypeStruct((NW, CR, D), jnp.float32)),
    mesh=plsc.VectorSubcoreMesh(core_axis_name="c", subcore_axis_name="s"),
    compiler_params=pltpu.CompilerParams(needs_layout_passes=False),
    scratch_types=[
        pltpu.VMEM((B,), jnp.int32),
        pltpu.VMEM((B,), jnp.int32),
        pltpu.VMEM((B,), jnp.int32),
        pltpu.VMEM((B,), jnp.int32),
        pltpu.VMEM((B, D), jnp.float32),
        pltpu.VMEM((B, D), jnp.float32),
        pltpu.VMEM((B, D), jnp.float32),
        pltpu.VMEM((CR, D), jnp.float32),
        pltpu.VMEM_SHARED((N_PAD, D), jnp.float32),
        pltpu.SemaphoreType.DMA,
        pltpu.SemaphoreType.DMA,
        pltpu.SemaphoreType.DMA,
        pltpu.SemaphoreType.DMA,
        pltpu.SemaphoreType.DMA,
    ],
)


def _fin_body(x_ref, mp_ref, cp_ref, wux_ref, wun_ref, o_ref):
    ssum = mp_ref[0] + mp_ref[1]
    cnt = jnp.sum(cp_ref[...], axis=0)[:, None]
    h = ssum / jnp.maximum(cnt, 1.0)
    y = (jnp.dot(x_ref[...], wux_ref[...], preferred_element_type=jnp.float32)
         + jnp.dot(h, wun_ref[...], preferred_element_type=jnp.float32))
    o_ref[...] = _leaky(y)


def _final(x_pad, msg_p, cnt_r, wu_x, wu_n):
    nb = 20
    bn = N_PAD // nb
    return pl.pallas_call(
        _fin_body,
        grid=(nb,),
        in_specs=[
            pl.BlockSpec((bn, D), lambda i: (i, 0)),
            pl.BlockSpec((NC, bn, D), lambda i: (0, i, 0)),
            pl.BlockSpec((NW, bn), lambda i: (0, i)),
            pl.BlockSpec((D, D), lambda i: (0, 0)),
            pl.BlockSpec((D, D), lambda i: (0, 0)),
        ],
        out_specs=pl.BlockSpec((bn, D), lambda i: (i, 0)),
        out_shape=jax.ShapeDtypeStruct((N_PAD, D), jnp.float32),
    )(x_pad, msg_p, cnt_r, wu_x, wu_n)


def kernel(x, edge_index, edge_attr, W_M, W_U):
    src = edge_index[0]
    dst = edge_index[1]
    w_s = W_M[:D]
    w_d = W_M[D:2 * D]
    w_e = W_M[2 * D:]
    wu_x = W_U[:D]
    wu_n = W_U[D:]

    xs, xd = _proj(x, w_s, w_d)
    ea = _ea_proj(edge_attr, w_e)

    msg_p, cnt_p = _edge_call(src, dst, xs, xd, ea)
    cnt_r = cnt_p.reshape(NW, N_PAD)

    x_pad = jnp.pad(x, ((0, N_PAD - N), (0, 0)))
    out = _final(x_pad, msg_p, cnt_r, wu_x, wu_n)
    return out[:N]
